# baseline (speedup 1.0000x reference)
import functools

import jax
import jax.numpy as jnp
from jax import lax
from jax.experimental import pallas as pl
from jax.experimental.pallas import tpu as pltpu
from jax.experimental.pallas import tpu_sc as plsc

VOCAB = 1000000
DIM = 64
BATCH = 4096
SEQ = 200

NUM_WORKERS = 32
ROWS = BATCH * SEQ
PER_W = ROWS // NUM_WORKERS
CHUNK = 800
NCHUNK = PER_W // CHUNK
SUB = 80
NSUB = CHUNK // SUB
GRP = 16
NGRP = SUB // GRP


def _pos_encoding():
    pos = jnp.arange(SEQ, dtype=jnp.float32)
    denom = 10000.0 ** jnp.linspace(0.0, 1.0, DIM)
    arg = pos[:, None] / denom[None, :]
    col = jnp.arange(DIM)
    return jnp.where(col[None, :] % 2 == 0, jnp.sin(arg), jnp.cos(arg))


def _body(idx_hbm, soff_hbm, table_hbm, pos_hbm, out_hbm,
          idx_v, soff_v, stg0, stg1, out_v, pos_v, sem0, sem1):
    wid = lax.axis_index("s") * 2 + lax.axis_index("c")
    base = wid * PER_W

    pltpu.sync_copy(pos_hbm, pos_v)
    stgs = (stg0, stg1)
    sems = (sem0, sem1)

    def chunk_body(c, carry):
        r0 = pl.multiple_of(base + c * CHUNK, CHUNK)
        pltpu.sync_copy(idx_hbm.at[pl.ds(r0, CHUNK)], idx_v)
        pltpu.sync_copy(soff_hbm.at[pl.ds(r0, CHUNK)], soff_v)

        def fire(j):
            return pltpu.async_copy(
                table_hbm.at[idx_v.at[pl.ds(j * SUB, SUB)]],
                stgs[j % 2], sems[j % 2])

        cp = fire(0)
        for j in range(NSUB):
            cp.wait()
            if j + 1 < NSUB:
                cp = fire(j + 1)
            stg = stgs[j % 2]

            def grp_body(g, carry2, j=j, stg=stg):
                gb = j * SUB + g * GRP
                lb = g * GRP
                ob = gb // 2
                sv = soff_v[pl.ds(gb, GRP)]
                for k in range(GRP):
                    r = gb + k
                    l = lb + k
                    p = lax.rem(r, SEQ)
                    s0 = sv[k]
                    d0 = (k % 2) * DIM
                    for q in range(4):
                        out_v[ob + k // 2, pl.ds(d0 + q * 16, 16)] = (
                            stg[l, pl.ds(s0 + q * 16, 16)]
                            + pos_v[p, pl.ds(q * 16, 16)])
                return carry2

            lax.fori_loop(0, NGRP, grp_body, 0)

        o0 = pl.multiple_of(r0 // 2, CHUNK // 2)
        pltpu.sync_copy(out_v, out_hbm.at[pl.ds(o0, CHUNK // 2)])
        return carry

    lax.fori_loop(0, NCHUNK, chunk_body, 0)


@jax.jit
def _run(idxp, soff, table2, pos):
    mesh = plsc.VectorSubcoreMesh(core_axis_name="c", subcore_axis_name="s")
    f = functools.partial(
        pl.kernel,
        mesh=mesh,
        out_type=jax.ShapeDtypeStruct((ROWS // 2, 2 * DIM), jnp.float32),
        scratch_types=[
            pltpu.VMEM((CHUNK,), jnp.int32),
            pltpu.VMEM((CHUNK,), jnp.int32),
            pltpu.VMEM((SUB, 2 * DIM), jnp.float32),
            pltpu.VMEM((SUB, 2 * DIM), jnp.float32),
            pltpu.VMEM((CHUNK // 2, 2 * DIM), jnp.float32),
            pltpu.VMEM((SEQ, DIM), jnp.float32),
            pltpu.SemaphoreType.DMA,
            pltpu.SemaphoreType.DMA,
        ],
        compiler_params=pltpu.CompilerParams(use_tc_tiling_on_sc=True),
    )(_body)
    return f(idxp, soff, table2, pos)


def kernel(input, table):
    flat = input.reshape(ROWS)
    idxp = flat >> 1
    soff = (flat & 1) * DIM
    table2 = table.reshape(VOCAB // 2, 2 * DIM)
    pos = _pos_encoding()
    out = _run(idxp, soff, table2, pos)
    return out.reshape(BATCH, SEQ, DIM)


# speedup vs baseline: 1.4364x; 1.4364x over previous
import functools

import jax
import jax.numpy as jnp
from jax import lax
from jax.experimental import pallas as pl
from jax.experimental.pallas import tpu as pltpu
from jax.experimental.pallas import tpu_sc as plsc

VOCAB = 1000000
DIM = 64
BATCH = 4096
SEQ = 200

NUM_WORKERS = 32
ROWS = BATCH * SEQ
PER_W = ROWS // NUM_WORKERS
CHUNK = 800
NCHUNK = PER_W // CHUNK
SUB = 100
NSUB = CHUNK // SUB
REPS = CHUNK // SEQ
BPC = CHUNK // SEQ


def _pos_encoding():
    pos = jnp.arange(SEQ, dtype=jnp.float32)
    denom = 10000.0 ** jnp.linspace(0.0, 1.0, DIM)
    arg = pos[:, None] / denom[None, :]
    col = jnp.arange(DIM)
    return jnp.where(col[None, :] % 2 == 0, jnp.sin(arg), jnp.cos(arg))


def _body(idx_hbm, table_hbm, pos_hbm, out_hbm,
          idx0, idx1, rows0, rows1, pos_v,
          sg0, sg1, so0, so1):
    wid = lax.axis_index("s") * 2 + lax.axis_index("c")
    base_sub = wid * (PER_W // SUB)

    pltpu.sync_copy(pos_hbm, pos_v)

    idxs = (idx0, idx1)
    rows = (rows0, rows1)
    sgs = (sg0, sg1)
    sos = (so0, so1)

    def fire_gathers(c):
        b = c % 2
        r0 = base_sub + c * NSUB
        pltpu.sync_copy(idx_hbm.at[pl.ds(r0, NSUB)], idxs[b])
        return [
            pltpu.async_copy(
                table_hbm.at[idxs[b].at[j]],
                rows[b].at[j // 2, pl.ds((j % 2) * SUB, SUB)],
                sgs[b],
            )
            for j in range(NSUB)
        ]

    def add_chunk(c):
        b = c % 2
        rv = rows[b]

        def add_row(r, carry2):
            p0 = pos_v[r, pl.ds(0, 16)]
            p1 = pos_v[r, pl.ds(16, 16)]
            p2 = pos_v[r, pl.ds(32, 16)]
            p3 = pos_v[r, pl.ds(48, 16)]
            for rep in range(REPS):
                rv[rep, r, pl.ds(0, 16)] = rv[rep, r, pl.ds(0, 16)] + p0
                rv[rep, r, pl.ds(16, 16)] = rv[rep, r, pl.ds(16, 16)] + p1
                rv[rep, r, pl.ds(32, 16)] = rv[rep, r, pl.ds(32, 16)] + p2
                rv[rep, r, pl.ds(48, 16)] = rv[rep, r, pl.ds(48, 16)] + p3
            return carry2

        lax.fori_loop(0, SEQ, add_row, 0)

    def writeback(c):
        b = c % 2
        b0 = wid * (PER_W // SEQ) + c * BPC
        return pltpu.async_copy(rows[b], out_hbm.at[pl.ds(b0, BPC)], sos[b])

    gathers = {0: fire_gathers(0)}
    out_copies = {}
    for c in range(NCHUNK):
        if c + 1 < NCHUNK:
            if c - 1 >= 0:
                out_copies.pop(c - 1).wait()
            gathers[c + 1] = fire_gathers(c + 1)
        for cp in gathers.pop(c):
            cp.wait()
        add_chunk(c)
        out_copies[c] = writeback(c)
    out_copies.pop(NCHUNK - 2).wait()
    out_copies.pop(NCHUNK - 1).wait()


@jax.jit
def _run(idx2d, table, pos):
    mesh = plsc.VectorSubcoreMesh(core_axis_name="c", subcore_axis_name="s")
    f = functools.partial(
        pl.kernel,
        mesh=mesh,
        out_type=jax.ShapeDtypeStruct((BATCH, SEQ, DIM), jnp.float32),
        scratch_types=[
            pltpu.VMEM((NSUB, SUB), jnp.int32),
            pltpu.VMEM((NSUB, SUB), jnp.int32),
            pltpu.VMEM((BPC, SEQ, DIM), jnp.float32),
            pltpu.VMEM((BPC, SEQ, DIM), jnp.float32),
            pltpu.VMEM((SEQ, DIM), jnp.float32),
            pltpu.SemaphoreType.DMA,
            pltpu.SemaphoreType.DMA,
            pltpu.SemaphoreType.DMA,
            pltpu.SemaphoreType.DMA,
        ],
        compiler_params=pltpu.CompilerParams(use_tc_tiling_on_sc=False),
    )(_body)
    return f(idx2d, table, pos)


def kernel(input, table):
    idx2d = input.reshape(ROWS // SUB, SUB)
    pos = _pos_encoding()
    return _run(idx2d, table, pos)


# speedup vs baseline: 1.4452x; 1.0061x over previous
import functools

import jax
import jax.numpy as jnp
from jax import lax
from jax.experimental import pallas as pl
from jax.experimental.pallas import tpu as pltpu
from jax.experimental.pallas import tpu_sc as plsc

VOCAB = 1000000
DIM = 64
BATCH = 4096
SEQ = 200

NUM_WORKERS = 32
ROWS = BATCH * SEQ
PER_W = ROWS // NUM_WORKERS
CHUNK = 400
NCHUNK = PER_W // CHUNK
SUB = 100
NSUB = CHUNK // SUB
REPS = CHUNK // SEQ
BPC = CHUNK // SEQ
IDX_ROWS = PER_W // SUB
NBUF = 3


def _pos_encoding():
    pos = jnp.arange(SEQ, dtype=jnp.float32)
    denom = 10000.0 ** jnp.linspace(0.0, 1.0, DIM)
    arg = pos[:, None] / denom[None, :]
    col = jnp.arange(DIM)
    return jnp.where(col[None, :] % 2 == 0, jnp.sin(arg), jnp.cos(arg))


def _body(idx_hbm, table_hbm, pos_hbm, out_hbm,
          idx_v, rows0, rows1, rows2, pos_v,
          sg0, sg1, sg2, so0, so1, so2):
    wid = lax.axis_index("s") * 2 + lax.axis_index("c")
    base_sub = wid * IDX_ROWS

    pltpu.sync_copy(pos_hbm, pos_v)
    pltpu.sync_copy(idx_hbm.at[pl.ds(base_sub, IDX_ROWS)], idx_v)

    rows = (rows0, rows1, rows2)
    sgs = (sg0, sg1, sg2)
    sos = (so0, so1, so2)

    def fire_gathers(c):
        b = c % NBUF
        return [
            pltpu.async_copy(
                table_hbm.at[idx_v.at[c * NSUB + j]],
                rows[b].at[j // 2, pl.ds((j % 2) * SUB, SUB)],
                sgs[b],
            )
            for j in range(NSUB)
        ]

    def add_chunk(c):
        rv = rows[c % NBUF]

        def add_row(r, carry2):
            p0 = pos_v[r, pl.ds(0, 16)]
            p1 = pos_v[r, pl.ds(16, 16)]
            p2 = pos_v[r, pl.ds(32, 16)]
            p3 = pos_v[r, pl.ds(48, 16)]
            for rep in range(REPS):
                rv[rep, r, pl.ds(0, 16)] = rv[rep, r, pl.ds(0, 16)] + p0
                rv[rep, r, pl.ds(16, 16)] = rv[rep, r, pl.ds(16, 16)] + p1
                rv[rep, r, pl.ds(32, 16)] = rv[rep, r, pl.ds(32, 16)] + p2
                rv[rep, r, pl.ds(48, 16)] = rv[rep, r, pl.ds(48, 16)] + p3
            return carry2

        lax.fori_loop(0, SEQ, add_row, 0)

    def writeback(c):
        b = c % NBUF
        b0 = wid * (PER_W // SEQ) + c * BPC
        return pltpu.async_copy(rows[b], out_hbm.at[pl.ds(b0, BPC)], sos[b])

    gathers = {0: fire_gathers(0), 1: fire_gathers(1)}
    out_copies = {}
    for c in range(NCHUNK):
        if c + 2 < NCHUNK:
            if c - 1 >= 0:
                out_copies.pop(c - 1).wait()
            gathers[c + 2] = fire_gathers(c + 2)
        for cp in gathers.pop(c):
            cp.wait()
        add_chunk(c)
        out_copies[c] = writeback(c)
    for c in (NCHUNK - 3, NCHUNK - 2, NCHUNK - 1):
        out_copies.pop(c).wait()


@jax.jit
def _run(idx2d, table, pos):
    mesh = plsc.VectorSubcoreMesh(core_axis_name="c", subcore_axis_name="s")
    f = functools.partial(
        pl.kernel,
        mesh=mesh,
        out_type=jax.ShapeDtypeStruct((BATCH, SEQ, DIM), jnp.float32),
        scratch_types=[
            pltpu.VMEM((IDX_ROWS, SUB), jnp.int32),
            pltpu.VMEM((BPC, SEQ, DIM), jnp.float32),
            pltpu.VMEM((BPC, SEQ, DIM), jnp.float32),
            pltpu.VMEM((BPC, SEQ, DIM), jnp.float32),
            pltpu.VMEM((SEQ, DIM), jnp.float32),
            pltpu.SemaphoreType.DMA,
            pltpu.SemaphoreType.DMA,
            pltpu.SemaphoreType.DMA,
            pltpu.SemaphoreType.DMA,
            pltpu.SemaphoreType.DMA,
            pltpu.SemaphoreType.DMA,
        ],
        compiler_params=pltpu.CompilerParams(use_tc_tiling_on_sc=False),
    )(_body)
    return f(idx2d, table, pos)


def kernel(input, table):
    idx2d = input.reshape(ROWS // SUB, SUB)
    pos = _pos_encoding()
    return _run(idx2d, table, pos)


# speedup vs baseline: 1.4466x; 1.0009x over previous
import functools

import jax
import jax.numpy as jnp
from jax import lax
from jax.experimental import pallas as pl
from jax.experimental.pallas import tpu as pltpu
from jax.experimental.pallas import tpu_sc as plsc

VOCAB = 1000000
DIM = 64
BATCH = 4096
SEQ = 200

NUM_WORKERS = 32
ROWS = BATCH * SEQ
PER_W = ROWS // NUM_WORKERS
CHUNK = 400
NCHUNK = PER_W // CHUNK
REPS = CHUNK // SEQ
NBUF = 3


def _pos_encoding():
    pos = jnp.arange(SEQ, dtype=jnp.float32)
    denom = 10000.0 ** jnp.linspace(0.0, 1.0, DIM)
    arg = pos[:, None] / denom[None, :]
    col = jnp.arange(DIM)
    return jnp.where(col[None, :] % 2 == 0, jnp.sin(arg), jnp.cos(arg))


def _body(idx_hbm, table_hbm, pos_hbm, out_hbm,
          idx_v, rows0, rows1, rows2, pos_v,
          sg0, sg1, sg2, so0, so1, so2):
    wid = lax.axis_index("s") * 2 + lax.axis_index("c")
    base = wid * PER_W

    pltpu.sync_copy(pos_hbm, pos_v)
    pltpu.sync_copy(idx_hbm.at[pl.ds(base, PER_W)], idx_v)

    rows = (rows0, rows1, rows2)
    sgs = (sg0, sg1, sg2)
    sos = (so0, so1, so2)

    def fire_gather(c):
        b = c % NBUF
        return pltpu.async_copy(
            table_hbm.at[idx_v.at[pl.ds(c * CHUNK, CHUNK)]],
            rows[b],
            sgs[b],
        )

    def add_chunk(c):
        rv = rows[c % NBUF]

        def add_row(r, carry2):
            p0 = pos_v[r, pl.ds(0, 16)]
            p1 = pos_v[r, pl.ds(16, 16)]
            p2 = pos_v[r, pl.ds(32, 16)]
            p3 = pos_v[r, pl.ds(48, 16)]
            for rep in range(REPS):
                q = rep * SEQ + r
                rv[q, pl.ds(0, 16)] = rv[q, pl.ds(0, 16)] + p0
                rv[q, pl.ds(16, 16)] = rv[q, pl.ds(16, 16)] + p1
                rv[q, pl.ds(32, 16)] = rv[q, pl.ds(32, 16)] + p2
                rv[q, pl.ds(48, 16)] = rv[q, pl.ds(48, 16)] + p3
            return carry2

        lax.fori_loop(0, SEQ, add_row, 0)

    def writeback(c):
        b = c % NBUF
        return pltpu.async_copy(
            rows[b], out_hbm.at[pl.ds(base + c * CHUNK, CHUNK)], sos[b])

    gathers = {0: fire_gather(0), 1: fire_gather(1)}
    out_copies = {}
    for c in range(NCHUNK):
        if c + 2 < NCHUNK:
            if c - 1 >= 0:
                out_copies.pop(c - 1).wait()
            gathers[c + 2] = fire_gather(c + 2)
        gathers.pop(c).wait()
        add_chunk(c)
        out_copies[c] = writeback(c)
    for c in (NCHUNK - 3, NCHUNK - 2, NCHUNK - 1):
        out_copies.pop(c).wait()


@jax.jit
def _run(idx_flat, table, pos):
    mesh = plsc.VectorSubcoreMesh(core_axis_name="c", subcore_axis_name="s")
    f = functools.partial(
        pl.kernel,
        mesh=mesh,
        out_type=jax.ShapeDtypeStruct((ROWS, DIM), jnp.float32),
        scratch_types=[
            pltpu.VMEM((PER_W,), jnp.int32),
            pltpu.VMEM((CHUNK, DIM), jnp.float32),
            pltpu.VMEM((CHUNK, DIM), jnp.float32),
            pltpu.VMEM((CHUNK, DIM), jnp.float32),
            pltpu.VMEM((SEQ, DIM), jnp.float32),
            pltpu.SemaphoreType.DMA,
            pltpu.SemaphoreType.DMA,
            pltpu.SemaphoreType.DMA,
            pltpu.SemaphoreType.DMA,
            pltpu.SemaphoreType.DMA,
            pltpu.SemaphoreType.DMA,
        ],
        compiler_params=pltpu.CompilerParams(use_tc_tiling_on_sc=False),
    )(_body)
    return f(idx_flat, table, pos).reshape(BATCH, SEQ, DIM)


def kernel(input, table):
    idx_flat = input.reshape(ROWS)
    pos = _pos_encoding()
    return _run(idx_flat, table, pos)
